# Initial kernel scaffold; baseline (speedup 1.0000x reference)
import jax
import jax.numpy as jnp
from jax.experimental import pallas as pl


def kernel(time, emb_month, emb_weekday):
    raise NotImplementedError("write your pallas kernel here")


# speedup vs baseline: 3.3570x; 3.3570x over previous
import math

import jax
import jax.numpy as jnp
from jax import lax
from jax.experimental import pallas as pl
from jax.experimental.pallas import tpu as pltpu
from jax.experimental.pallas import tpu_sc as plsc

B, L = 16384, 200
N = B * L
D_OUT = 54
D_PAD = 64
NUM_WORKERS = 32
BLK = 128
TOK_PER_WORKER = N // NUM_WORKERS
ITERS = TOK_PER_WORKER // BLK
LANES = 16


def _sc_kernel(table_hbm, time_hbm, feat_hbm, out_hbm,
               time_v, idx_v, gath_v, fcol_v, feat_v, sem):
    cid = lax.axis_index("c")
    sid = lax.axis_index("s")
    wid = cid * 16 + sid

    pltpu.sync_copy(feat_hbm, feat_v)

    lane = lax.iota(jnp.int32, LANES)

    def body(i, carry):
        tok0 = (wid * ITERS + i) * BLK
        pltpu.sync_copy(time_hbm.at[pl.ds(tok0, BLK), :], time_v)

        for j in range(BLK // LANES):
            rows = lane + (j * LANES)
            month = plsc.load_gather(time_v, [rows, jnp.zeros((LANES,), jnp.int32)])
            wday = plsc.load_gather(time_v, [rows, jnp.full((LANES,), 4, jnp.int32)])
            idx = (month.astype(jnp.int32) - 1) * 8 + wday.astype(jnp.int32)
            idx_v[pl.ds(j * LANES, LANES)] = idx

        pltpu.async_copy(table_hbm.at[idx_v], gath_v, sem).wait()
        gath_dma = pltpu.make_async_copy(
            gath_v, out_hbm.at[pl.ds(tok0, BLK), 0:48], sem)
        gath_dma.start()

        for j in range(BLK // LANES):
            rows = lane + (j * LANES)
            day = plsc.load_gather(time_v, [rows, jnp.full((LANES,), 1, jnp.int32)])
            hour = plsc.load_gather(time_v, [rows, jnp.full((LANES,), 2, jnp.int32)])
            minute = plsc.load_gather(time_v, [rows, jnp.full((LANES,), 3, jnp.int32)])
            di = day.astype(jnp.int32)
            hi = hour.astype(jnp.int32) + 31
            mi = minute.astype(jnp.int32) + 55
            for col, fidx in ((0, di), (2, hi), (4, mi)):
                for c in range(2):
                    val = plsc.load_gather(feat_v, [fidx, jnp.full((LANES,), c, jnp.int32)])
                    plsc.store_scatter(fcol_v, [rows, jnp.full((LANES,), col + c, jnp.int32)], val)

        pltpu.sync_copy(fcol_v, out_hbm.at[pl.ds(tok0, BLK), 48:54])
        gath_dma.wait()
        return carry

    lax.fori_loop(0, ITERS, body, 0)


def kernel(time, emb_month, emb_weekday):
    m_ids = jnp.arange(96, dtype=jnp.int32) // 8
    w_ids = jnp.arange(96, dtype=jnp.int32) % 8
    table = jnp.concatenate([emb_month[m_ids], emb_weekday[w_ids]], axis=1)

    d = jnp.arange(31, dtype=jnp.float32) * (2 * math.pi / 31)
    h = jnp.arange(24, dtype=jnp.float32) * (2 * math.pi / 24)
    m = jnp.arange(60, dtype=jnp.float32) * (2 * math.pi / 60)
    ang = jnp.concatenate([d, h, m, jnp.zeros((5,), jnp.float32)])
    feat = jnp.stack([jnp.sin(ang), jnp.cos(ang)], axis=1)

    time2 = time.reshape(N, 5)

    mesh = plsc.VectorSubcoreMesh(core_axis_name="c", subcore_axis_name="s")
    out = pl.kernel(
        _sc_kernel,
        mesh=mesh,
        out_type=jax.ShapeDtypeStruct((N, D_OUT), jnp.float32),
        scratch_types=[
            pltpu.VMEM((BLK, 5), jnp.float32),
            pltpu.VMEM((BLK,), jnp.int32),
            pltpu.VMEM((BLK, 48), jnp.float32),
            pltpu.VMEM((BLK, 6), jnp.float32),
            pltpu.VMEM((120, 2), jnp.float32),
            pltpu.SemaphoreType.DMA,
        ],
        compiler_params=pltpu.CompilerParams(
            needs_layout_passes=False, use_tc_tiling_on_sc=False),
    )(table, time2, feat)
    return out.reshape(B, L, D_OUT)
